# baseline (speedup 1.0000x reference)
import jax
import jax.numpy as jnp
from jax import lax
from jax.experimental import pallas as pl
from jax.experimental.pallas import tpu as pltpu
from jax.experimental.pallas import tpu_sc as plsc

VOCAB = 30522
HIDDEN = 768
MAX_POS = 512
BATCH = 128
SEQ = 512

L = 16
NC = 2
NS = 16
NW = NC * NS
P_PER_W = SEQ // NW
KH = HIDDEN // L
KH2 = KH // 2

_GDN = lax.GatherDimensionNumbers(
    offset_dims=(), collapsed_slice_dims=(0,), start_index_map=(0,))


def _permute(vec, idx):
    return lax.gather(vec, idx, _GDN, (1,),
                      mode=lax.GatherScatterMode.PROMISE_IN_BOUNDS)


def _rsqrt(x):
    i = plsc.bitcast(x, jnp.int32)
    i = jnp.int32(0x5F3759DF) - (i >> 1)
    y = plsc.bitcast(i, jnp.float32)
    for _ in range(3):
        y = y * (1.5 - 0.5 * x * y * y)
    return y


def _unpack2(v):
    vb = plsc.bitcast(v, jnp.bfloat16)
    return plsc.unpack(vb, format=plsc.PackFormat.INTERLEAVED,
                       preferred_element_type=jnp.float32)


def _body(ids_ref, tt_ref, word_ref, base_ref, gb_ref,
          out_ref, idx_v, tt_v, base_v, gb_v, sums_v, sumsq_v,
          eb_v, rows2, outb2, gsem, osem):
    cid = lax.axis_index("c")
    sid = lax.axis_index("s")
    wid = sid * NC + cid
    p0 = wid * P_PER_W

    pltpu.sync_copy(base_ref.at[0, pl.ds(p0, P_PER_W), :], base_v.at[0])
    pltpu.sync_copy(base_ref.at[1, pl.ds(p0, P_PER_W), :], base_v.at[1])
    pltpu.sync_copy(gb_ref, gb_v)
    pltpu.sync_copy(ids_ref.at[wid], idx_v)
    pltpu.sync_copy(tt_ref.at[wid], tt_v)

    inv_h = jnp.float32(1.0 / HIDDEN)
    zeros = jnp.zeros((L,), jnp.float32)
    lanes = lax.iota(jnp.int32, L)
    nacc = 4

    def _compute(b, rows, outb):
        ttrow = tt_v[b, :]

        def _ttj(j):
            return _permute(ttrow, jnp.full((L, 1), j, dtype=jnp.int32))[0]

        def _tokA(j, ttj):
            ttn = _ttj(jnp.minimum(j + 1, P_PER_W - 1))

            accs = [zeros] * nacc
            acc2s = [zeros] * nacc

            def _triple(k2):
                return (rows[j, pl.ds(k2 * 2 * L, L)],
                        rows[j, pl.ds((k2 * 2 + 1) * L, L)],
                        base_v[ttj, j, pl.ds(k2 * L, L)])
            ld = [_triple(0), _triple(1)]
            for k2 in range(KH2):
                if k2 + 2 < KH2:
                    ld.append(_triple(k2 + 2))
                r0, r1, pk = ld[k2]
                ba, bb = _unpack2(pk)
                e0 = r0 + ba
                e1 = r1 + bb
                ep = plsc.pack(e0, e1, format=plsc.PackFormat.INTERLEAVED,
                               preferred_element_type=jnp.bfloat16)
                eb_v[j, pl.ds(k2 * L, L)] = plsc.bitcast(ep, jnp.int32)
                a = (2 * k2) % nacc
                accs[a] = accs[a] + e0
                acc2s[a] = acc2s[a] + e0 * e0
                accs[a + 1] = accs[a + 1] + e1
                acc2s[a + 1] = acc2s[a + 1] + e1 * e1
            acc = (accs[0] + accs[1]) + (accs[2] + accs[3])
            acc2 = (acc2s[0] + acc2s[1]) + (acc2s[2] + acc2s[3])
            jcol = jnp.full((L,), j, dtype=jnp.int32)
            plsc.store_scatter(sums_v, [lanes, jcol], acc)
            plsc.store_scatter(sumsq_v, [lanes, jcol], acc2)
            return ttn
        lax.fori_loop(0, P_PER_W, _tokA, _ttj(0))

        rs = [sums_v[r, :] for r in range(L)]
        rq = [sumsq_v[r, :] for r in range(L)]
        while len(rs) > 1:
            rs = [rs[i] + rs[i + 1] for i in range(0, len(rs), 2)]
            rq = [rq[i] + rq[i + 1] for i in range(0, len(rq), 2)]
        mean_all = rs[0] * inv_h
        var_all = jnp.maximum(rq[0] * inv_h - mean_all * mean_all, 0.0)
        rstd_all = _rsqrt(var_all + 1e-12)
        off_all = -mean_all * rstd_all

        rstd_j = []
        off_j = []
        for j in range(P_PER_W):
            jf = jnp.full((L, 1), j, dtype=jnp.int32)
            rstd_j.append(_permute(rstd_all, jf))
            off_j.append(_permute(off_all, jf))

        @plsc.parallel_loop(0, KH2, unroll=1)
        def _kb(k2):
            g0, b0 = _unpack2(gb_v[pl.ds(2 * k2 * L, L)])
            g1, b1 = _unpack2(gb_v[pl.ds((2 * k2 + 1) * L, L)])
            for j in range(P_PER_W):
                e0, e1 = _unpack2(eb_v[j, pl.ds(k2 * L, L)])
                outb[j, pl.ds(2 * k2 * L, L)] = (
                    (e0 * rstd_j[j] + off_j[j]) * g0 + b0)
                outb[j, pl.ds((2 * k2 + 1) * L, L)] = (
                    (e1 * rstd_j[j] + off_j[j]) * g1 + b1)

    def _gather(b, rows):
        return pltpu.async_copy(word_ref.at[idx_v.at[b]], rows, gsem)

    def _gwait(b, rows):
        pltpu.make_async_copy(word_ref.at[idx_v.at[b]], rows, gsem).wait()

    def _ostart(b, outb):
        pltpu.async_copy(outb, out_ref.at[b, pl.ds(p0, P_PER_W), :], osem)

    def _owait(b, outb):
        pltpu.make_async_copy(
            outb, out_ref.at[b, pl.ds(p0, P_PER_W), :], osem).wait()

    _gather(0, rows2.at[0])

    def _bb(i, _):
        for ph in range(2):
            b = 2 * i + ph
            cur = rows2.at[ph]
            outb = outb2.at[ph]

            @pl.when(b + 1 < BATCH)
            def _():
                _gather(b + 1, rows2.at[(ph + 1) % 2])

            _gwait(b, cur)

            @pl.when(b >= 2)
            def _():
                _owait(b - 2, outb)

            _compute(b, cur, outb)
            _ostart(b, outb)
        return 0
    lax.fori_loop(0, BATCH // 2, _bb, 0)

    _owait(BATCH - 2, outb2.at[0])
    _owait(BATCH - 1, outb2.at[1])


def kernel(input_ids, token_type_ids, word_emb, pos_emb, type_emb,
           ln_gamma, ln_beta):
    ids_r = input_ids.astype(jnp.int32).reshape(BATCH, NW, P_PER_W)
    ids_r = ids_r.transpose(1, 0, 2)
    tt_r = token_type_ids.astype(jnp.int32).reshape(BATCH, NW, P_PER_W)
    tt_r = tt_r.transpose(1, 0, 2)

    base = pos_emb[None, :, :] + type_emb[:, None, :]
    base = base.astype(jnp.bfloat16).reshape(2, SEQ, KH2, 2, L)
    base = base.transpose(0, 1, 2, 4, 3).reshape(2, SEQ, HIDDEN // 2, 2)
    base = lax.bitcast_convert_type(base, jnp.int32)

    gb = jnp.stack([ln_gamma.reshape(KH, L), ln_beta.reshape(KH, L)],
                   axis=-1).astype(jnp.bfloat16).reshape(HIDDEN, 2)
    gb = lax.bitcast_convert_type(gb, jnp.int32)

    mesh = plsc.VectorSubcoreMesh(core_axis_name="c", subcore_axis_name="s")
    f = pl.kernel(
        _body,
        out_type=jax.ShapeDtypeStruct((BATCH, SEQ, HIDDEN), jnp.float32),
        mesh=mesh,
        compiler_params=pltpu.CompilerParams(needs_layout_passes=False),
        scratch_types=[
            pltpu.VMEM((BATCH, P_PER_W), jnp.int32),
            pltpu.VMEM((BATCH, P_PER_W), jnp.int32),
            pltpu.VMEM((2, P_PER_W, HIDDEN // 2), jnp.int32),
            pltpu.VMEM((HIDDEN,), jnp.int32),
            pltpu.VMEM((L, P_PER_W), jnp.float32),
            pltpu.VMEM((L, P_PER_W), jnp.float32),
            pltpu.VMEM((P_PER_W, HIDDEN // 2), jnp.int32),
            pltpu.VMEM((2, P_PER_W, HIDDEN), jnp.float32),
            pltpu.VMEM((2, P_PER_W, HIDDEN), jnp.float32),
            pltpu.SemaphoreType.DMA,
            pltpu.SemaphoreType.DMA,
        ],
    )
    return f(ids_r, tt_r, word_emb, base, gb)


# speedup vs baseline: 1.1570x; 1.1570x over previous
import jax
import jax.numpy as jnp
from jax import lax
from jax.experimental import pallas as pl
from jax.experimental.pallas import tpu as pltpu
from jax.experimental.pallas import tpu_sc as plsc

VOCAB = 30522
HIDDEN = 768
MAX_POS = 512
BATCH = 128
SEQ = 512

L = 16
NC = 2
NS = 16
NW = NC * NS
P_PER_W = SEQ // NW
KH = HIDDEN // L
KH2 = KH // 2

_GDN = lax.GatherDimensionNumbers(
    offset_dims=(), collapsed_slice_dims=(0,), start_index_map=(0,))


def _permute(vec, idx):
    return lax.gather(vec, idx, _GDN, (1,),
                      mode=lax.GatherScatterMode.PROMISE_IN_BOUNDS)


def _rsqrt(x):
    i = plsc.bitcast(x, jnp.int32)
    i = jnp.int32(0x5F3759DF) - (i >> 1)
    y = plsc.bitcast(i, jnp.float32)
    for _ in range(3):
        y = y * (1.5 - 0.5 * x * y * y)
    return y


def _unpack2(v):
    vb = plsc.bitcast(v, jnp.bfloat16)
    return plsc.unpack(vb, format=plsc.PackFormat.INTERLEAVED,
                       preferred_element_type=jnp.float32)


def _body(ids_ref, tt_ref, word_ref, base_ref, gb_ref,
          out_ref, idx_v, tt_v, base_v, gb_v, sums_v, sumsq_v,
          rows2, outb2, gsem, osem):
    cid = lax.axis_index("c")
    sid = lax.axis_index("s")
    wid = sid * NC + cid
    p0 = wid * P_PER_W

    pltpu.sync_copy(base_ref.at[0, pl.ds(p0, P_PER_W), :], base_v.at[0])
    pltpu.sync_copy(base_ref.at[1, pl.ds(p0, P_PER_W), :], base_v.at[1])
    pltpu.sync_copy(gb_ref, gb_v)
    pltpu.sync_copy(ids_ref.at[wid], idx_v)
    pltpu.sync_copy(tt_ref.at[wid], tt_v)

    inv_h = jnp.float32(1.0 / HIDDEN)
    zeros = jnp.zeros((L,), jnp.float32)
    lanes = lax.iota(jnp.int32, L)
    nacc = 4

    def _compute(b, rows, outb):
        ttrow = tt_v[b, :]

        def _ttj(j):
            return _permute(ttrow, jnp.full((L, 1), j, dtype=jnp.int32))[0]

        def _tokA(j, ttj):
            ttn = _ttj(jnp.minimum(j + 1, P_PER_W - 1))

            accs = [zeros] * nacc
            acc2s = [zeros] * nacc

            def _triple(k2):
                return (rows[j, pl.ds(k2 * 2 * L, L)],
                        rows[j, pl.ds((k2 * 2 + 1) * L, L)],
                        base_v[ttj, j, pl.ds(k2 * L, L)])
            ld = [_triple(0), _triple(1)]
            for k2 in range(KH2):
                if k2 + 2 < KH2:
                    ld.append(_triple(k2 + 2))
                r0, r1, pk = ld[k2]
                ba, bb = _unpack2(pk)
                e0 = r0 + ba
                e1 = r1 + bb
                outb[j, pl.ds(k2 * 2 * L, L)] = e0
                outb[j, pl.ds((k2 * 2 + 1) * L, L)] = e1
                a = (2 * k2) % nacc
                accs[a] = accs[a] + e0
                acc2s[a] = acc2s[a] + e0 * e0
                accs[a + 1] = accs[a + 1] + e1
                acc2s[a + 1] = acc2s[a + 1] + e1 * e1
            acc = (accs[0] + accs[1]) + (accs[2] + accs[3])
            acc2 = (acc2s[0] + acc2s[1]) + (acc2s[2] + acc2s[3])
            jcol = jnp.full((L,), j, dtype=jnp.int32)
            plsc.store_scatter(sums_v, [lanes, jcol], acc)
            plsc.store_scatter(sumsq_v, [lanes, jcol], acc2)
            return ttn
        lax.fori_loop(0, P_PER_W, _tokA, _ttj(0))

        rs = [sums_v[r, :] for r in range(L)]
        rq = [sumsq_v[r, :] for r in range(L)]
        while len(rs) > 1:
            rs = [rs[i] + rs[i + 1] for i in range(0, len(rs), 2)]
            rq = [rq[i] + rq[i + 1] for i in range(0, len(rq), 2)]
        mean_all = rs[0] * inv_h
        var_all = jnp.maximum(rq[0] * inv_h - mean_all * mean_all, 0.0)
        rstd_all = _rsqrt(var_all + 1e-12)
        off_all = -mean_all * rstd_all

        rstd_j = []
        off_j = []
        for j in range(P_PER_W):
            jf = jnp.full((L, 1), j, dtype=jnp.int32)
            rstd_j.append(_permute(rstd_all, jf))
            off_j.append(_permute(off_all, jf))

        @plsc.parallel_loop(0, KH, unroll=1)
        def _kb(k):
            sl = pl.ds(k * L, L)
            g, be = _unpack2(gb_v[sl])
            for j in range(P_PER_W):
                e = outb[j, sl]
                outb[j, sl] = (e * rstd_j[j] + off_j[j]) * g + be

    def _gather(b, rows):
        return pltpu.async_copy(word_ref.at[idx_v.at[b]], rows, gsem)

    def _gwait(b, rows):
        pltpu.make_async_copy(word_ref.at[idx_v.at[b]], rows, gsem).wait()

    def _ostart(b, outb):
        pltpu.async_copy(outb, out_ref.at[b, pl.ds(p0, P_PER_W), :], osem)

    def _owait(b, outb):
        pltpu.make_async_copy(
            outb, out_ref.at[b, pl.ds(p0, P_PER_W), :], osem).wait()

    _gather(0, rows2.at[0])

    def _bb(i, _):
        for ph in range(2):
            b = 2 * i + ph
            cur = rows2.at[ph]
            outb = outb2.at[ph]

            @pl.when(b + 1 < BATCH)
            def _():
                _gather(b + 1, rows2.at[(ph + 1) % 2])

            _gwait(b, cur)

            @pl.when(b >= 2)
            def _():
                _owait(b - 2, outb)

            _compute(b, cur, outb)
            _ostart(b, outb)
        return 0
    lax.fori_loop(0, BATCH // 2, _bb, 0)

    _owait(BATCH - 2, outb2.at[0])
    _owait(BATCH - 1, outb2.at[1])


def kernel(input_ids, token_type_ids, word_emb, pos_emb, type_emb,
           ln_gamma, ln_beta):
    ids_r = input_ids.astype(jnp.int32).reshape(BATCH, NW, P_PER_W)
    ids_r = ids_r.transpose(1, 0, 2)
    tt_r = token_type_ids.astype(jnp.int32).reshape(BATCH, NW, P_PER_W)
    tt_r = tt_r.transpose(1, 0, 2)

    base = pos_emb[None, :, :] + type_emb[:, None, :]
    base = base.astype(jnp.bfloat16).reshape(2, SEQ, KH2, 2, L)
    base = base.transpose(0, 1, 2, 4, 3).reshape(2, SEQ, HIDDEN // 2, 2)
    base = lax.bitcast_convert_type(base, jnp.int32)

    gb = jnp.stack([ln_gamma.reshape(KH, L), ln_beta.reshape(KH, L)],
                   axis=-1).astype(jnp.bfloat16).reshape(HIDDEN, 2)
    gb = lax.bitcast_convert_type(gb, jnp.int32)

    mesh = plsc.VectorSubcoreMesh(core_axis_name="c", subcore_axis_name="s")
    f = pl.kernel(
        _body,
        out_type=jax.ShapeDtypeStruct((BATCH, SEQ, HIDDEN), jnp.float32),
        mesh=mesh,
        compiler_params=pltpu.CompilerParams(needs_layout_passes=False),
        scratch_types=[
            pltpu.VMEM((BATCH, P_PER_W), jnp.int32),
            pltpu.VMEM((BATCH, P_PER_W), jnp.int32),
            pltpu.VMEM((2, P_PER_W, HIDDEN // 2), jnp.int32),
            pltpu.VMEM((HIDDEN,), jnp.int32),
            pltpu.VMEM((L, P_PER_W), jnp.float32),
            pltpu.VMEM((L, P_PER_W), jnp.float32),
            pltpu.VMEM((2, P_PER_W, HIDDEN), jnp.float32),
            pltpu.VMEM((2, P_PER_W, HIDDEN), jnp.float32),
            pltpu.SemaphoreType.DMA,
            pltpu.SemaphoreType.DMA,
        ],
    )
    return f(ids_r, tt_r, word_emb, base, gb)
